# Initial kernel scaffold; baseline (speedup 1.0000x reference)
import jax
import jax.numpy as jnp
from jax.experimental import pallas as pl


def kernel(logits, input_ids, top_k):
    raise NotImplementedError("write your pallas kernel here")


# speedup vs baseline: 77.1808x; 77.1808x over previous
import functools

import jax
import jax.numpy as jnp
from jax import lax
from jax.experimental import pallas as pl
from jax.experimental.pallas import tpu as pltpu
from jax.experimental.pallas import tpu_sc as plsc

TEMP = 0.7
TOP_P = 0.9
REP = 1.1
PAD = -3.0e38

L = 16
CAP = 256
BUF = CAP + 32
NBUF = BUF // L
KEEP = 64
CAND = 128
NCAND = CAND // L
ROWS_PER_SUBCORE = 2


def _body(logits_hbm, ids_hbm, tk_hbm, out_hbm, tok_hbm,
          row_v, ids_v, pen_v, bufv, bufi, wv, civ, spv, cmv, tkv, tokv):
    V = row_v.shape[0]
    T = ids_v.shape[0]
    NV = V // L
    NT = T // L
    wid = lax.axis_index("s") * 2 + lax.axis_index("c")
    iota = lax.iota(jnp.int32, L)
    pltpu.sync_copy(tk_hbm, tkv)
    tk = tkv[...][0]

    def _count_ge(t, ptr):
        def cg(k, acc):
            v = bufv[pl.ds(k * L, L)]
            valid = (iota + k * L) < ptr
            return acc + jnp.sum(jnp.where(valid & (v >= t), 1, 0).astype(jnp.int32))
        return lax.fori_loop(0, NBUF, cg, jnp.int32(0))

    def _prune(c):
        ptr, _thr = c

        def mm(k, mc):
            mn, mx = mc
            v = bufv[pl.ds(k * L, L)]
            valid = (iota + k * L) < ptr
            mn = jnp.minimum(mn, jnp.min(jnp.where(valid, v, jnp.float32(3e38))))
            mx = jnp.maximum(mx, jnp.max(jnp.where(valid, v, jnp.float32(-3e38))))
            return mn, mx
        mn, mx = lax.fori_loop(0, NBUF, mm,
                               (jnp.float32(3e38), jnp.float32(-3e38)))
        span = jnp.maximum(mx - mn, jnp.float32(1e-30))

        def bis(_, lh):
            lo, hi = lh
            mid = lo + (hi - lo) * jnp.float32(0.5)
            ok = _count_ge(mid, ptr) >= KEEP
            return jnp.where(ok, mid, lo), jnp.where(ok, hi, mid)
        lo, _ = lax.fori_loop(0, 40, bis, (mn, mx + span))

        def comp(k, cp):
            v = bufv[pl.ds(k * L, L)]
            ii = bufi[pl.ds(k * L, L)]
            m = ((iota + k * L) < ptr) & (v >= lo)
            pos = cp + plsc.cumsum(m.astype(jnp.int32)) - 1
            plsc.store_scatter(bufv, [pos], v, mask=m)
            plsc.store_scatter(bufi, [pos], ii, mask=m)
            return cp + jnp.sum(m.astype(jnp.int32))
        ptr2 = lax.fori_loop(0, NBUF, comp, jnp.int32(0))
        return jnp.minimum(ptr2, jnp.int32(CAP - 1)), lo

    for r in range(ROWS_PER_SUBCORE):
        row = wid * ROWS_PER_SUBCORE + r
        pltpu.sync_copy(logits_hbm.at[row], row_v)
        pltpu.sync_copy(ids_hbm.at[row], ids_v)

        def pgather(k, _):
            iv = ids_v[pl.ds(k * L, L)]
            g = plsc.load_gather(row_v, [iv])
            pen_v[pl.ds(k * L, L)] = jnp.where(g < 0, g * REP, g / REP)
            return 0
        lax.fori_loop(0, NT, pgather, 0)

        def pscatter(k, _):
            iv = ids_v[pl.ds(k * L, L)]
            plsc.store_scatter(row_v, [iv], pen_v[pl.ds(k * L, L)])
            return 0
        lax.fori_loop(0, NT, pscatter, 0)

        def scan_body(i, c):
            ptr, thr = c
            v = row_v[pl.ds(i * L, L)]
            hit = jnp.max(jnp.where(v >= thr, 1, 0).astype(jnp.int32))

            def app(c2):
                p2, t2 = lax.cond(c2[0] >= CAP - L, _prune, lambda x: x, c2)
                m = v >= t2
                pos = p2 + plsc.cumsum(m.astype(jnp.int32)) - 1
                plsc.store_scatter(bufv, [pos], v, mask=m)
                plsc.store_scatter(bufi, [pos], iota + i * L, mask=m)
                return p2 + jnp.sum(m.astype(jnp.int32)), t2
            return lax.cond(hit > 0, app, lambda c2: c2, c)
        ptr, thr = lax.fori_loop(0, NV, scan_body,
                                 (jnp.int32(0), jnp.float32(PAD)))

        ptr, thr = lax.cond(ptr > CAND, _prune, lambda c: c, (ptr, thr))
        nf = jnp.minimum(ptr, jnp.int32(CAND))
        for k in range(NCAND):
            lane = iota + k * L
            valid = lane < nf
            v = bufv[pl.ds(k * L, L)]
            ii = bufi[pl.ds(k * L, L)]
            wv[pl.ds(k * L, L)] = jnp.where(valid, v / TEMP, jnp.float32(PAD))
            civ[pl.ds(k * L, L)] = jnp.where(valid, ii, jnp.int32(0))

        def rk(j, accs):
            s = wv[pl.ds(j, L)][0]
            out = []
            for k in range(NCAND):
                wk = wv[pl.ds(k * L, L)]
                lane = iota + k * L
                beat = (s > wk) | ((s == wk) & (j < lane))
                out.append(accs[k] + jnp.where(beat, 1, 0).astype(jnp.int32))
            return tuple(out)
        ranks = lax.fori_loop(0, CAND, rk,
                              tuple(jnp.zeros((L,), jnp.int32)
                                    for _ in range(NCAND)))

        v50 = jnp.float32(0.0)
        mx = jnp.float32(0.0)
        ntok = jnp.int32(0)
        for k in range(NCAND):
            wk = wv[pl.ds(k * L, L)]
            ck = civ[pl.ds(k * L, L)]
            v50 += jnp.sum(jnp.where(ranks[k] == tk - 1, wk, jnp.float32(0.0)))
            mx += jnp.sum(jnp.where(ranks[k] == 0, wk, jnp.float32(0.0)))
            ntok += jnp.sum(jnp.where(ranks[k] == 0, ck, 0))

        es = []
        den = jnp.float32(0.0)
        for k in range(NCAND):
            wk = wv[pl.ds(k * L, L)]
            e = jnp.where(wk >= v50, jnp.exp(wk - mx), jnp.float32(0.0))
            es.append(e)
            den += jnp.sum(e)
        for k in range(NCAND):
            plsc.store_scatter(spv, [ranks[k]], es[k] / den)

        carry = jnp.float32(0.0)
        for k in range(NCAND):
            v = spv[pl.ds(k * L, L)]
            cum = plsc.cumsum(v) + carry
            cmv[pl.ds(k * L, L)] = cum
            carry = jnp.sum(jnp.where(iota == L - 1, cum, jnp.float32(0.0)))

        e2s = []
        den2 = jnp.float32(0.0)
        for k in range(NCAND):
            wk = wv[pl.ds(k * L, L)]
            cum = plsc.load_gather(cmv, [ranks[k]])
            keep = (wk >= v50) & ((ranks[k] == 0) | (cum <= TOP_P))
            e2 = jnp.where(keep, es[k], jnp.float32(0.0))
            e2s.append(e2)
            den2 += jnp.sum(e2)

        zero = jnp.zeros((L,), jnp.float32)
        def zb(k, _):
            row_v[pl.ds(k * L, L)] = zero
            return 0
        lax.fori_loop(0, NV, zb, 0)
        for k in range(NCAND):
            pf = e2s[k] / den2
            plsc.store_scatter(row_v, [civ[pl.ds(k * L, L)]], pf,
                               mask=pf > jnp.float32(0.0))
        pltpu.sync_copy(row_v, out_hbm.at[row])
        tokv[...] = jnp.full((L,), ntok, dtype=jnp.int32)
        pltpu.sync_copy(tokv, tok_hbm.at[row])


def kernel(logits, input_ids, top_k):
    B, V = logits.shape
    T = input_ids.shape[1]
    tk_arr = jnp.full((L,), top_k, dtype=jnp.int32)
    mesh = plsc.VectorSubcoreMesh(core_axis_name="c", subcore_axis_name="s")
    run = pl.kernel(
        _body,
        out_type=(
            jax.ShapeDtypeStruct((B, V), jnp.float32),
            jax.ShapeDtypeStruct((B, L), jnp.int32),
        ),
        mesh=mesh,
        compiler_params=pltpu.CompilerParams(needs_layout_passes=False),
        scratch_types=[
            pltpu.VMEM((V,), jnp.float32),
            pltpu.VMEM((T,), jnp.int32),
            pltpu.VMEM((T,), jnp.float32),
            pltpu.VMEM((BUF,), jnp.float32),
            pltpu.VMEM((BUF,), jnp.int32),
            pltpu.VMEM((CAND + L,), jnp.float32),
            pltpu.VMEM((CAND,), jnp.int32),
            pltpu.VMEM((CAND,), jnp.float32),
            pltpu.VMEM((CAND,), jnp.float32),
            pltpu.VMEM((L,), jnp.int32),
            pltpu.VMEM((L,), jnp.int32),
        ],
    )
    probs, tok = run(logits, input_ids.astype(jnp.int32), tk_arr)
    return probs, tok[:, 0]
